# Initial kernel scaffold; baseline (speedup 1.0000x reference)
import jax
import jax.numpy as jnp
from jax.experimental import pallas as pl


def kernel(pts, fts, params):
    raise NotImplementedError("write your pallas kernel here")


# speedup vs baseline: 4.8861x; 4.8861x over previous
import numpy as np
import jax
import jax.numpy as jnp
from jax.experimental import pallas as pl

_NUM_CLASS = 40
_N_PTS = 1024
_LAYER_CFG = [(3, 32, 8, 1, -1), (32, 64, 8, 2, -1), (64, 96, 8, 4, -1),
              (96, 128, 12, 4, 120), (128, 160, 12, 6, 120)]
_SAMPLE_IDX = np.random.RandomState(123).choice(_N_PTS, 120, replace=False)


def _relu(x):
    return jnp.maximum(x, 0.0)


def _dot(a, b):
    return jax.lax.dot_general(a, b, (((1,), (0,)), ((), ())),
                               preferred_element_type=jnp.float32)


def _derived(cfg):
    C_in, C_out, K, D, _ = cfg
    Cmid, Cx = C_out // 4, C_out // 2
    Cm = Cmid + Cx
    dm = min(int(np.ceil(C_out / C_in)), 4)
    iters = (K - 1) * D + 2
    return Cmid, Cx, Cm, dm, iters


def _prep_weights(params, i, cfg, fc):
    p = params["pcnn%d" % i]
    C_in, C_out, K, D, _ = cfg
    Cmid, Cx, Cm, dm, _ = _derived(cfg)
    ws = [
        p["dense1"]["W"].T, p["dense1"]["b"][None],
        p["dense2"]["W"].T, p["dense2"]["b"][None],
        jnp.transpose(p["xconv_w"], (2, 1, 0)).reshape(3 * K, K * K),
        p["xconv_b"][None],
        p["xd1"]["W"].T, p["xd1"]["b"][None],
        p["xd2"]["W"].T, p["xd2"]["b"][None],
        jnp.transpose(p["dw_w"], (2, 1, 0)).reshape(K * dm, Cm),
        jnp.transpose(p["pw_w"].reshape(C_out, Cm, dm), (2, 1, 0)).reshape(dm * Cm, C_out),
        (p["dw_b"] @ p["pw_w"].T)[None],
        (p["bn_g"] / np.sqrt(1.0 + 1e-5))[None],
        p["bn_b"][None],
    ]
    if i == 0:
        ws += [p["dense"]["W"].T, p["dense"]["b"][None]]
    if not fc:
        nxt = params["pcnn%d" % (i + 1)]["dense"]
        ws += [nxt["W"].T, nxt["b"][None]]
    else:
        ws += [params["fc1"]["W"].T, params["fc1"]["b"][None],
               params["fc2"]["W"].T, params["fc2"]["b"][None],
               params["fc3"]["W"].T, params["fc3"]["b"][None]]
    return [w.astype(jnp.float32) for w in ws]


def _make_body(cfg, R, N, l0, fc):
    C_in, C_out, K, D, _ = cfg
    Cmid, Cx, Cm, dm, iters = _derived(cfg)
    K2 = K * K

    def body(*refs):
        ptsT = refs[0][0]
        pts = refs[1][0]
        rep = refs[2][0]
        ftsin = refs[3][0]
        wv = [r[...] for r in refs[4:-1]]
        out_ref = refs[-1]
        (w1t, b1, w2t, b2, wxf, xb, xd1t, xb1, xd2t, xb2,
         dwt2, pwd, pwb, sv, bbv) = wv[:15]
        rest = wv[15:]
        if l0:
            w0t, b0 = rest[0], rest[1]
            rest = rest[2:]
            fts1 = _relu(_dot(ftsin, w0t) + b0)
        else:
            fts1 = ftsin

        pts2 = jnp.sum(ptsT * ptsT, axis=0, keepdims=True)
        rep2 = jnp.sum(rep * rep, axis=1, keepdims=True)
        d2 = rep2 + pts2 - 2.0 * _dot(rep, ptsT)
        iota = jax.lax.broadcasted_iota(jnp.int32, (R, N), 1)

        p_list, f_list = [], []
        for r in range(iters):
            vmin = jnp.min(d2, axis=1, keepdims=True)
            idxc = jnp.where(d2 == vmin, iota, N)
            fidx = jnp.min(idxc, axis=1, keepdims=True)
            onehot = idxc == fidx
            if r >= 1 and (r - 1) % D == 0:
                oh = onehot.astype(jnp.float32)
                p_list.append(_dot(oh, pts))
                f_list.append(_dot(oh, fts1))
            if r != iters - 1:
                d2 = jnp.where(onehot, jnp.float32(np.inf), d2)

        pl_j = [pj - rep for pj in p_list]
        PLs = jnp.concatenate(pl_j, axis=0)
        PLl = jnp.concatenate(pl_j, axis=1)

        fl = _relu(_dot(PLs, w1t) + b1)
        fl = _relu(_dot(fl, w2t) + b2)

        X = _relu(_dot(PLl, wxf) + xb)
        X = _relu(_dot(X, xd1t) + xb1)
        X = _dot(X, xd2t) + xb2

        cat = [jnp.concatenate([fl[j * R:(j + 1) * R], f_list[j]], axis=1)
               for j in range(K)]

        acc = [jnp.zeros((R, Cm), jnp.float32) for _ in range(dm)]
        for k in range(K):
            fx = jnp.zeros((R, Cm), jnp.float32)
            for j in range(K):
                c = k * K + j
                fx = fx + X[:, c:c + 1] * cat[j]
            for dd in range(dm):
                acc[dd] = acc[dd] + fx * dwt2[k * dm + dd:k * dm + dd + 1, :]

        y = jnp.zeros((R, C_out), jnp.float32) + pwb
        for dd in range(dm):
            y = y + _dot(acc[dd], pwd[dd * Cm:(dd + 1) * Cm, :])
        y = _relu(y) * sv + bbv

        if not fc:
            wet, eb = rest
            out_ref[0] = _relu(_dot(y, wet) + eb)
        else:
            f1t, f1b, f2t, f2b, f3t, f3b = rest
            h = _relu(_dot(y, f1t) + f1b)
            h = _relu(_dot(h, f2t) + f2b)
            lg = _dot(h, f3t) + f3b
            m = jnp.sum(lg, axis=0, keepdims=True) * (1.0 / R)
            out_ref[0] = jnp.broadcast_to(m, (8, _NUM_CLASS))

    return body


def _layer_call(i, cfg, pts, ptsT, rep, fts1, params):
    B = pts.shape[0]
    N = pts.shape[1]
    Nrep = rep.shape[1]
    fc = (i == len(_LAYER_CFG) - 1)
    l0 = (i == 0)
    R = 128 if Nrep == _N_PTS else Nrep
    T = Nrep // R
    Cf = fts1.shape[2]
    weights = _prep_weights(params, i, cfg, fc)
    if fc:
        out_shape = (B, 8, _NUM_CLASS)
        out_block = (1, 8, _NUM_CLASS)
    else:
        C_next = _LAYER_CFG[i + 1][1] // 2
        out_shape = (B, Nrep, C_next)
        out_block = (1, R, C_next)

    def _const(b, t):
        return (0, 0)

    in_specs = [
        pl.BlockSpec((1, 3, N), lambda b, t: (b, 0, 0)),
        pl.BlockSpec((1, N, 3), lambda b, t: (b, 0, 0)),
        pl.BlockSpec((1, R, 3), lambda b, t: (b, t, 0)),
        pl.BlockSpec((1, N, Cf), lambda b, t: (b, 0, 0)),
    ] + [pl.BlockSpec(w.shape, _const) for w in weights]

    body = _make_body(cfg, R, N, l0, fc)
    return pl.pallas_call(
        body,
        grid=(B, T),
        in_specs=in_specs,
        out_specs=pl.BlockSpec(out_block, lambda b, t: (b, t, 0)),
        out_shape=jax.ShapeDtypeStruct(out_shape, jnp.float32),
    )(ptsT, pts, rep, fts1, *weights)


def kernel(pts, fts, params):
    pts = pts.astype(jnp.float32)
    ptsT = jnp.transpose(pts, (0, 2, 1))
    rep3 = pts[:, _SAMPLE_IDX, :]
    rep3T = jnp.transpose(rep3, (0, 2, 1))

    fts1 = _layer_call(0, _LAYER_CFG[0], pts, ptsT, pts, fts.astype(jnp.float32), params)
    fts1 = _layer_call(1, _LAYER_CFG[1], pts, ptsT, pts, fts1, params)
    fts1 = _layer_call(2, _LAYER_CFG[2], pts, ptsT, pts, fts1, params)
    fts1 = _layer_call(3, _LAYER_CFG[3], pts, ptsT, rep3, fts1, params)
    out = _layer_call(4, _LAYER_CFG[4], rep3, rep3T, rep3, fts1, params)
    return out[:, 0, :]


# speedup vs baseline: 8.3729x; 1.7136x over previous
import numpy as np
import jax
import jax.numpy as jnp
from jax.experimental import pallas as pl

_NUM_CLASS = 40
_N_PTS = 1024
_LAYER_CFG = [(3, 32, 8, 1, -1), (32, 64, 8, 2, -1), (64, 96, 8, 4, -1),
              (96, 128, 12, 4, 120), (128, 160, 12, 6, 120)]
_SAMPLE_IDX = np.random.RandomState(123).choice(_N_PTS, 120, replace=False)


def _relu(x):
    return jnp.maximum(x, 0.0)


def _dot(a, b):
    return jax.lax.dot_general(a, b, (((1,), (0,)), ((), ())),
                               preferred_element_type=jnp.float32)


def _derived(cfg):
    C_in, C_out, K, D, _ = cfg
    Cmid, Cx = C_out // 4, C_out // 2
    Cm = Cmid + Cx
    dm = min(int(np.ceil(C_out / C_in)), 4)
    iters = (K - 1) * D + 2
    return Cmid, Cx, Cm, dm, iters


def _prep_weights(params, i, cfg, fc):
    p = params["pcnn%d" % i]
    C_in, C_out, K, D, _ = cfg
    Cmid, Cx, Cm, dm, _ = _derived(cfg)
    ws = [
        p["dense1"]["W"].T, p["dense1"]["b"][None],
        p["dense2"]["W"].T, p["dense2"]["b"][None],
        jnp.transpose(p["xconv_w"], (2, 1, 0)).reshape(3 * K, K * K),
        p["xconv_b"][None],
        p["xd1"]["W"].T, p["xd1"]["b"][None],
        p["xd2"]["W"].T, p["xd2"]["b"][None],
        jnp.transpose(p["dw_w"], (2, 1, 0)).reshape(K * dm, Cm),
        jnp.transpose(p["pw_w"].reshape(C_out, Cm, dm), (2, 1, 0)).reshape(dm * Cm, C_out),
        (p["dw_b"] @ p["pw_w"].T)[None],
        (p["bn_g"] / np.sqrt(1.0 + 1e-5))[None],
        p["bn_b"][None],
    ]
    if i == 0:
        ws += [p["dense"]["W"].T, p["dense"]["b"][None]]
    if not fc:
        nxt = params["pcnn%d" % (i + 1)]["dense"]
        ws += [nxt["W"].T, nxt["b"][None]]
    else:
        ws += [params["fc1"]["W"].T, params["fc1"]["b"][None],
               params["fc2"]["W"].T, params["fc2"]["b"][None],
               params["fc3"]["W"].T, params["fc3"]["b"][None]]
    return [w.astype(jnp.float32) for w in ws]


def _make_body(cfg, R, N, l0, fc):
    C_in, C_out, K, D, _ = cfg
    Cmid, Cx, Cm, dm, iters = _derived(cfg)
    K2 = K * K

    def body(*refs):
        ptsT = refs[0][0]
        pts = refs[1][0]
        rep = refs[2][0]
        ftsin = refs[3][0]
        wv = [r[...] for r in refs[4:-1]]
        out_ref = refs[-1]
        (w1t, b1, w2t, b2, wxf, xb, xd1t, xb1, xd2t, xb2,
         dwt2, pwd, pwb, sv, bbv) = wv[:15]
        rest = wv[15:]
        if l0:
            w0t, b0 = rest[0], rest[1]
            rest = rest[2:]
            fts1 = _relu(_dot(ftsin, w0t) + b0)
        else:
            fts1 = ftsin

        pts2 = jnp.sum(ptsT * ptsT, axis=0, keepdims=True)
        rep2 = jnp.sum(rep * rep, axis=1, keepdims=True)
        d2 = rep2 + pts2 - 2.0 * _dot(rep, ptsT)
        iota = jax.lax.broadcasted_iota(jnp.int32, (R, N), 1).astype(jnp.float32)
        big = jnp.float32(N)

        p_list, f_list = [], []
        for r in range(iters):
            vmin = jnp.min(d2, axis=1, keepdims=True)
            idxc = jnp.where(d2 == vmin, iota, big)
            fidx = jnp.min(idxc, axis=1, keepdims=True)
            onehot = idxc == fidx
            if r >= 1 and (r - 1) % D == 0:
                oh = jnp.where(onehot, 1.0, 0.0)
                p_list.append(_dot(oh, pts))
                f_list.append(_dot(oh, fts1))
            if r != iters - 1:
                d2 = jnp.where(onehot, jnp.float32(np.inf), d2)

        pl_j = [pj - rep for pj in p_list]
        PLs = jnp.concatenate(pl_j, axis=0)
        PLl = jnp.concatenate(pl_j, axis=1)

        fl = _relu(_dot(PLs, w1t) + b1)
        fl = _relu(_dot(fl, w2t) + b2)

        X = _relu(_dot(PLl, wxf) + xb)
        X = _relu(_dot(X, xd1t) + xb1)
        X = _dot(X, xd2t) + xb2

        cat = [jnp.concatenate([fl[j * R:(j + 1) * R], f_list[j]], axis=1)
               for j in range(K)]

        acc = [jnp.zeros((R, Cm), jnp.float32) for _ in range(dm)]
        for k in range(K):
            fx = jnp.zeros((R, Cm), jnp.float32)
            for j in range(K):
                c = k * K + j
                fx = fx + X[:, c:c + 1] * cat[j]
            for dd in range(dm):
                acc[dd] = acc[dd] + fx * dwt2[k * dm + dd:k * dm + dd + 1, :]

        y = jnp.zeros((R, C_out), jnp.float32) + pwb
        for dd in range(dm):
            y = y + _dot(acc[dd], pwd[dd * Cm:(dd + 1) * Cm, :])
        y = _relu(y) * sv + bbv

        if not fc:
            wet, eb = rest
            out_ref[0] = _relu(_dot(y, wet) + eb)
        else:
            f1t, f1b, f2t, f2b, f3t, f3b = rest
            h = _relu(_dot(y, f1t) + f1b)
            h = _relu(_dot(h, f2t) + f2b)
            lg = _dot(h, f3t) + f3b
            m = jnp.sum(lg, axis=0, keepdims=True) * (1.0 / R)
            out_ref[0] = jnp.broadcast_to(m, (8, _NUM_CLASS))

    return body


def _layer_call(i, cfg, pts, ptsT, rep, fts1, params):
    B = pts.shape[0]
    N = pts.shape[1]
    Nrep = rep.shape[1]
    fc = (i == len(_LAYER_CFG) - 1)
    l0 = (i == 0)
    R = 512 if Nrep == _N_PTS else Nrep
    T = Nrep // R
    Cf = fts1.shape[2]
    weights = _prep_weights(params, i, cfg, fc)
    if fc:
        out_shape = (B, 8, _NUM_CLASS)
        out_block = (1, 8, _NUM_CLASS)
    else:
        C_next = _LAYER_CFG[i + 1][1] // 2
        out_shape = (B, Nrep, C_next)
        out_block = (1, R, C_next)

    def _const(b, t):
        return (0, 0)

    in_specs = [
        pl.BlockSpec((1, 3, N), lambda b, t: (b, 0, 0)),
        pl.BlockSpec((1, N, 3), lambda b, t: (b, 0, 0)),
        pl.BlockSpec((1, R, 3), lambda b, t: (b, t, 0)),
        pl.BlockSpec((1, N, Cf), lambda b, t: (b, 0, 0)),
    ] + [pl.BlockSpec(w.shape, _const) for w in weights]

    body = _make_body(cfg, R, N, l0, fc)
    return pl.pallas_call(
        body,
        grid=(B, T),
        in_specs=in_specs,
        out_specs=pl.BlockSpec(out_block, lambda b, t: (b, t, 0)),
        out_shape=jax.ShapeDtypeStruct(out_shape, jnp.float32),
    )(ptsT, pts, rep, fts1, *weights)


def kernel(pts, fts, params):
    pts = pts.astype(jnp.float32)
    ptsT = jnp.transpose(pts, (0, 2, 1))
    rep3 = pts[:, _SAMPLE_IDX, :]
    rep3T = jnp.transpose(rep3, (0, 2, 1))

    fts1 = _layer_call(0, _LAYER_CFG[0], pts, ptsT, pts, fts.astype(jnp.float32), params)
    fts1 = _layer_call(1, _LAYER_CFG[1], pts, ptsT, pts, fts1, params)
    fts1 = _layer_call(2, _LAYER_CFG[2], pts, ptsT, pts, fts1, params)
    fts1 = _layer_call(3, _LAYER_CFG[3], pts, ptsT, rep3, fts1, params)
    out = _layer_call(4, _LAYER_CFG[4], rep3, rep3T, rep3, fts1, params)
    return out[:, 0, :]


# speedup vs baseline: 9.5872x; 1.1450x over previous
import numpy as np
import jax
import jax.numpy as jnp
from jax.experimental import pallas as pl

_NUM_CLASS = 40
_N_PTS = 1024
_LAYER_CFG = [(3, 32, 8, 1, -1), (32, 64, 8, 2, -1), (64, 96, 8, 4, -1),
              (96, 128, 12, 4, 120), (128, 160, 12, 6, 120)]
_SAMPLE_IDX = np.random.RandomState(123).choice(_N_PTS, 120, replace=False)


def _relu(x):
    return jnp.maximum(x, 0.0)


def _dot(a, b):
    return jax.lax.dot_general(a, b, (((1,), (0,)), ((), ())),
                               preferred_element_type=jnp.float32)


def _derived(cfg):
    C_in, C_out, K, D, _ = cfg
    Cmid, Cx = C_out // 4, C_out // 2
    Cm = Cmid + Cx
    dm = min(int(np.ceil(C_out / C_in)), 4)
    iters = (K - 1) * D + 2
    return Cmid, Cx, Cm, dm, iters


def _prep_weights(params, i, cfg, fc):
    p = params["pcnn%d" % i]
    C_in, C_out, K, D, _ = cfg
    Cmid, Cx, Cm, dm, _ = _derived(cfg)
    ws = [
        p["dense1"]["W"].T, p["dense1"]["b"][None],
        p["dense2"]["W"].T, p["dense2"]["b"][None],
        jnp.transpose(p["xconv_w"], (2, 1, 0)).reshape(3 * K, K * K),
        p["xconv_b"][None],
        p["xd1"]["W"].T, p["xd1"]["b"][None],
        p["xd2"]["W"].T, p["xd2"]["b"][None],
        jnp.transpose(p["dw_w"], (2, 1, 0)).reshape(K * dm, Cm),
        jnp.transpose(p["pw_w"].reshape(C_out, Cm, dm), (2, 1, 0)).reshape(dm * Cm, C_out),
        (p["dw_b"] @ p["pw_w"].T)[None],
        (p["bn_g"] / np.sqrt(1.0 + 1e-5))[None],
        p["bn_b"][None],
    ]
    if i == 0:
        ws += [p["dense"]["W"].T, p["dense"]["b"][None]]
    if not fc:
        nxt = params["pcnn%d" % (i + 1)]["dense"]
        ws += [nxt["W"].T, nxt["b"][None]]
    else:
        ws += [params["fc1"]["W"].T, params["fc1"]["b"][None],
               params["fc2"]["W"].T, params["fc2"]["b"][None],
               params["fc3"]["W"].T, params["fc3"]["b"][None]]
    return [w.astype(jnp.float32) for w in ws]


_SEL_RANKS = 30
_SEL_COLS = 32


def _make_sel_body(R, N):
    def body(ptsT_ref, rep_ref, out_ref):
        ptsT = ptsT_ref[0]
        rep = rep_ref[0]
        pts2 = jnp.sum(ptsT * ptsT, axis=0, keepdims=True)
        rep2 = jnp.sum(rep * rep, axis=1, keepdims=True)
        d2 = rep2 + pts2 - 2.0 * _dot(rep, ptsT)
        iota = jax.lax.broadcasted_iota(jnp.int32, (R, N), 1).astype(jnp.float32)
        big = jnp.float32(N)
        cols = []
        for r in range(_SEL_RANKS):
            vmin = jnp.min(d2, axis=1, keepdims=True)
            idxc = jnp.where(d2 == vmin, iota, big)
            fidx = jnp.min(idxc, axis=1, keepdims=True)
            cols.append(fidx)
            if r != _SEL_RANKS - 1:
                d2 = jnp.where(idxc == fidx, jnp.float32(np.inf), d2)
        cols.append(jnp.zeros((R, _SEL_COLS - _SEL_RANKS), jnp.float32))
        out_ref[0] = jnp.concatenate(cols, axis=1)

    return body


def _make_body(cfg, R, N, l0, fc, use_sel):
    C_in, C_out, K, D, _ = cfg
    Cmid, Cx, Cm, dm, iters = _derived(cfg)
    K2 = K * K

    def body(*refs):
        if use_sel:
            pts = refs[0][0]
            rep = refs[1][0]
            ftsin = refs[2][0]
            idxa = refs[3][0]
            wv = [r[...] for r in refs[4:-1]]
        else:
            ptsT = refs[0][0]
            pts = refs[1][0]
            rep = refs[2][0]
            ftsin = refs[3][0]
            wv = [r[...] for r in refs[4:-1]]
        out_ref = refs[-1]
        (w1t, b1, w2t, b2, wxf, xb, xd1t, xb1, xd2t, xb2,
         dwt2, pwd, pwb, sv, bbv) = wv[:15]
        rest = wv[15:]
        if l0:
            w0t, b0 = rest[0], rest[1]
            rest = rest[2:]
            fts1 = _relu(_dot(ftsin, w0t) + b0)
        else:
            fts1 = ftsin

        iota = jax.lax.broadcasted_iota(jnp.int32, (R, N), 1).astype(jnp.float32)
        p_list, f_list = [], []
        if use_sel:
            for j in range(K):
                col = idxa[:, 1 + j * D:2 + j * D]
                oh = jnp.where(iota == col, 1.0, 0.0)
                p_list.append(_dot(oh, pts))
                f_list.append(_dot(oh, fts1))
        else:
            pts2 = jnp.sum(ptsT * ptsT, axis=0, keepdims=True)
            rep2 = jnp.sum(rep * rep, axis=1, keepdims=True)
            d2 = rep2 + pts2 - 2.0 * _dot(rep, ptsT)
            big = jnp.float32(N)
            for r in range(iters):
                vmin = jnp.min(d2, axis=1, keepdims=True)
                idxc = jnp.where(d2 == vmin, iota, big)
                fidx = jnp.min(idxc, axis=1, keepdims=True)
                onehot = idxc == fidx
                if r >= 1 and (r - 1) % D == 0:
                    oh = jnp.where(onehot, 1.0, 0.0)
                    p_list.append(_dot(oh, pts))
                    f_list.append(_dot(oh, fts1))
                if r != iters - 1:
                    d2 = jnp.where(onehot, jnp.float32(np.inf), d2)

        pl_j = [pj - rep for pj in p_list]
        PLs = jnp.concatenate(pl_j, axis=0)
        PLl = jnp.concatenate(pl_j, axis=1)

        fl = _relu(_dot(PLs, w1t) + b1)
        fl = _relu(_dot(fl, w2t) + b2)

        X = _relu(_dot(PLl, wxf) + xb)
        X = _relu(_dot(X, xd1t) + xb1)
        X = _dot(X, xd2t) + xb2

        cat = [jnp.concatenate([fl[j * R:(j + 1) * R], f_list[j]], axis=1)
               for j in range(K)]

        acc = [jnp.zeros((R, Cm), jnp.float32) for _ in range(dm)]
        for k in range(K):
            fx = jnp.zeros((R, Cm), jnp.float32)
            for j in range(K):
                c = k * K + j
                fx = fx + X[:, c:c + 1] * cat[j]
            for dd in range(dm):
                acc[dd] = acc[dd] + fx * dwt2[k * dm + dd:k * dm + dd + 1, :]

        y = jnp.zeros((R, C_out), jnp.float32) + pwb
        for dd in range(dm):
            y = y + _dot(acc[dd], pwd[dd * Cm:(dd + 1) * Cm, :])
        y = _relu(y) * sv + bbv

        if not fc:
            wet, eb = rest
            out_ref[0] = _relu(_dot(y, wet) + eb)
        else:
            f1t, f1b, f2t, f2b, f3t, f3b = rest
            h = _relu(_dot(y, f1t) + f1b)
            h = _relu(_dot(h, f2t) + f2b)
            lg = _dot(h, f3t) + f3b
            m = jnp.sum(lg, axis=0, keepdims=True) * (1.0 / R)
            out_ref[0] = jnp.broadcast_to(m, (8, _NUM_CLASS))

    return body


def _sel_call(pts, ptsT):
    B, N = pts.shape[0], pts.shape[1]
    R = 512
    body = _make_sel_body(R, N)
    return pl.pallas_call(
        body,
        grid=(B, N // R),
        in_specs=[
            pl.BlockSpec((1, 3, N), lambda b, t: (b, 0, 0)),
            pl.BlockSpec((1, R, 3), lambda b, t: (b, t, 0)),
        ],
        out_specs=pl.BlockSpec((1, R, _SEL_COLS), lambda b, t: (b, t, 0)),
        out_shape=jax.ShapeDtypeStruct((B, N, _SEL_COLS), jnp.float32),
    )(ptsT, pts)


def _layer_call(i, cfg, pts, ptsT, rep, fts1, params, sel=None):
    B = pts.shape[0]
    N = pts.shape[1]
    Nrep = rep.shape[1]
    fc = (i == len(_LAYER_CFG) - 1)
    l0 = (i == 0)
    R = 512 if Nrep == _N_PTS else Nrep
    T = Nrep // R
    Cf = fts1.shape[2]
    weights = _prep_weights(params, i, cfg, fc)
    if fc:
        out_shape = (B, 8, _NUM_CLASS)
        out_block = (1, 8, _NUM_CLASS)
    else:
        C_next = _LAYER_CFG[i + 1][1] // 2
        out_shape = (B, Nrep, C_next)
        out_block = (1, R, C_next)

    def _const(b, t):
        return (0, 0)

    if sel is not None:
        in_specs = [
            pl.BlockSpec((1, N, 3), lambda b, t: (b, 0, 0)),
            pl.BlockSpec((1, R, 3), lambda b, t: (b, t, 0)),
            pl.BlockSpec((1, N, Cf), lambda b, t: (b, 0, 0)),
            pl.BlockSpec((1, R, _SEL_COLS), lambda b, t: (b, t, 0)),
        ] + [pl.BlockSpec(w.shape, _const) for w in weights]
        args = (pts, rep, fts1, sel, *weights)
    else:
        in_specs = [
            pl.BlockSpec((1, 3, N), lambda b, t: (b, 0, 0)),
            pl.BlockSpec((1, N, 3), lambda b, t: (b, 0, 0)),
            pl.BlockSpec((1, R, 3), lambda b, t: (b, t, 0)),
            pl.BlockSpec((1, N, Cf), lambda b, t: (b, 0, 0)),
        ] + [pl.BlockSpec(w.shape, _const) for w in weights]
        args = (ptsT, pts, rep, fts1, *weights)

    body = _make_body(cfg, R, N, l0, fc, sel is not None)
    return pl.pallas_call(
        body,
        grid=(B, T),
        in_specs=in_specs,
        out_specs=pl.BlockSpec(out_block, lambda b, t: (b, t, 0)),
        out_shape=jax.ShapeDtypeStruct(out_shape, jnp.float32),
    )(*args)


def kernel(pts, fts, params):
    pts = pts.astype(jnp.float32)
    ptsT = jnp.transpose(pts, (0, 2, 1))
    rep3 = pts[:, _SAMPLE_IDX, :]
    rep3T = jnp.transpose(rep3, (0, 2, 1))

    sel = _sel_call(pts, ptsT)
    fts1 = _layer_call(0, _LAYER_CFG[0], pts, ptsT, pts, fts.astype(jnp.float32), params, sel)
    fts1 = _layer_call(1, _LAYER_CFG[1], pts, ptsT, pts, fts1, params, sel)
    fts1 = _layer_call(2, _LAYER_CFG[2], pts, ptsT, pts, fts1, params, sel)
    fts1 = _layer_call(3, _LAYER_CFG[3], pts, ptsT, rep3, fts1, params)
    out = _layer_call(4, _LAYER_CFG[4], rep3, rep3T, rep3, fts1, params)
    return out[:, 0, :]


# speedup vs baseline: 10.1641x; 1.0602x over previous
import numpy as np
import jax
import jax.numpy as jnp
from jax.experimental import pallas as pl

_NUM_CLASS = 40
_N_PTS = 1024
_LAYER_CFG = [(3, 32, 8, 1, -1), (32, 64, 8, 2, -1), (64, 96, 8, 4, -1),
              (96, 128, 12, 4, 120), (128, 160, 12, 6, 120)]
_SAMPLE_IDX = np.random.RandomState(123).choice(_N_PTS, 120, replace=False)

_SEL_RANKS = 30
_SEL_COLS = 32


def _relu(x):
    return jnp.maximum(x, 0.0)


def _dot(a, b):
    return jax.lax.dot_general(a, b, (((1,), (0,)), ((), ())),
                               preferred_element_type=jnp.float32)


def _derived(cfg):
    C_in, C_out, K, D, _ = cfg
    Cmid, Cx = C_out // 4, C_out // 2
    Cm = Cmid + Cx
    dm = min(int(np.ceil(C_out / C_in)), 4)
    iters = (K - 1) * D + 2
    return Cmid, Cx, Cm, dm, iters


def _prep_weights(params, i, cfg, fc):
    p = params["pcnn%d" % i]
    C_in, C_out, K, D, _ = cfg
    Cmid, Cx, Cm, dm, _ = _derived(cfg)
    ws = [
        p["dense1"]["W"].T, p["dense1"]["b"][None],
        p["dense2"]["W"].T, p["dense2"]["b"][None],
        jnp.transpose(p["xconv_w"], (2, 1, 0)).reshape(3 * K, K * K),
        p["xconv_b"][None],
        p["xd1"]["W"].T, p["xd1"]["b"][None],
        p["xd2"]["W"].T, p["xd2"]["b"][None],
        jnp.transpose(p["dw_w"], (2, 1, 0)).reshape(K * dm, Cm),
        jnp.transpose(p["pw_w"].reshape(C_out, Cm, dm), (2, 1, 0)).reshape(dm * Cm, C_out),
        (p["dw_b"] @ p["pw_w"].T)[None],
        (p["bn_g"] / np.sqrt(1.0 + 1e-5))[None],
        p["bn_b"][None],
    ]
    if i == 0:
        ws += [p["dense"]["W"].T, p["dense"]["b"][None]]
    if not fc:
        nxt = params["pcnn%d" % (i + 1)]["dense"]
        ws += [nxt["W"].T, nxt["b"][None]]
    else:
        ws += [params["fc1"]["W"].T, params["fc1"]["b"][None],
               params["fc2"]["W"].T, params["fc2"]["b"][None],
               params["fc3"]["W"].T, params["fc3"]["b"][None]]
    return [w.astype(jnp.float32) for w in ws]


def _make_body(cfg, R, N, G, l0, fc, use_sel, emit_sel):
    C_in, C_out, K, D, _ = cfg
    Cmid, Cx, Cm, dm, iters = _derived(cfg)
    K2 = K * K
    RG = R * G
    n_ext = _SEL_RANKS if emit_sel else iters

    def body(*refs):
        nout = 2 if emit_sel else 1
        if use_sel:
            pts_r, rep_r, fts_r, idx_r = refs[:4]
        else:
            ptsT_r, pts_r, rep_r, fts_r = refs[:4]
        wv = [r[...] for r in refs[4:len(refs) - nout]]
        if emit_sel:
            out_ref, sel_ref = refs[-2], refs[-1]
        else:
            out_ref = refs[-1]
        (w1t, b1, w2t, b2, wxf, xb, xd1t, xb1, xd2t, xb2,
         dwt2, pwd, pwb, sv, bbv) = wv[:15]
        rest = wv[15:]
        if l0:
            w0t, b0 = rest[0], rest[1]
            rest = rest[2:]

        fts1_g = []
        for g in range(G):
            f = fts_r[g]
            fts1_g.append(_relu(_dot(f, w0t) + b0) if l0 else f)

        iota = jax.lax.broadcasted_iota(jnp.int32, (RG, N), 1).astype(jnp.float32)
        rep = jnp.concatenate([rep_r[g] for g in range(G)], axis=0) if G > 1 \
            else rep_r[0]

        def gathers(oh):
            ps, fs = [], []
            for g in range(G):
                ohg = oh[g * R:(g + 1) * R] if G > 1 else oh
                ps.append(_dot(ohg, pts_r[g]))
                fs.append(_dot(ohg, fts1_g[g]))
            if G > 1:
                return jnp.concatenate(ps, 0), jnp.concatenate(fs, 0)
            return ps[0], fs[0]

        p_list, f_list = [], []
        if use_sel:
            idxa = idx_r[0]
            for j in range(K):
                col = idxa[:, 1 + j * D:2 + j * D]
                oh = jnp.where(iota == col, 1.0, 0.0)
                pj, fj = gathers(oh)
                p_list.append(pj)
                f_list.append(fj)
        else:
            d2s = []
            for g in range(G):
                ptsT = ptsT_r[g]
                repg = rep_r[g]
                pts2 = jnp.sum(ptsT * ptsT, axis=0, keepdims=True)
                rep2 = jnp.sum(repg * repg, axis=1, keepdims=True)
                d2s.append(rep2 + pts2 - 2.0 * _dot(repg, ptsT))
            d2 = jnp.concatenate(d2s, axis=0) if G > 1 else d2s[0]
            big = jnp.float32(N)
            sel_cols = []
            for r in range(n_ext):
                vmin = jnp.min(d2, axis=1, keepdims=True)
                idxc = jnp.where(d2 == vmin, iota, big)
                fidx = jnp.min(idxc, axis=1, keepdims=True)
                onehot = idxc == fidx
                if emit_sel:
                    sel_cols.append(fidx)
                if 1 <= r <= 1 + (K - 1) * D and (r - 1) % D == 0:
                    oh = jnp.where(onehot, 1.0, 0.0)
                    pj, fj = gathers(oh)
                    p_list.append(pj)
                    f_list.append(fj)
                if r != n_ext - 1:
                    d2 = jnp.where(onehot, jnp.float32(np.inf), d2)
            if emit_sel:
                sel_cols.append(jnp.zeros((RG, _SEL_COLS - _SEL_RANKS), jnp.float32))
                sel_ref[0] = jnp.concatenate(sel_cols, axis=1)

        pl_j = [pj - rep for pj in p_list]
        PLs = jnp.concatenate(pl_j, axis=0)
        PLl = jnp.concatenate(pl_j, axis=1)

        fl = _relu(_dot(PLs, w1t) + b1)
        fl = _relu(_dot(fl, w2t) + b2)

        X = _relu(_dot(PLl, wxf) + xb)
        X = _relu(_dot(X, xd1t) + xb1)
        X = _dot(X, xd2t) + xb2

        cat = [jnp.concatenate([fl[j * RG:(j + 1) * RG], f_list[j]], axis=1)
               for j in range(K)]

        acc = [jnp.zeros((RG, Cm), jnp.float32) for _ in range(dm)]
        for k in range(K):
            fx = jnp.zeros((RG, Cm), jnp.float32)
            for j in range(K):
                c = k * K + j
                fx = fx + X[:, c:c + 1] * cat[j]
            for dd in range(dm):
                acc[dd] = acc[dd] + fx * dwt2[k * dm + dd:k * dm + dd + 1, :]

        y = jnp.zeros((RG, C_out), jnp.float32) + pwb
        for dd in range(dm):
            y = y + _dot(acc[dd], pwd[dd * Cm:(dd + 1) * Cm, :])
        y = _relu(y) * sv + bbv

        if not fc:
            wet, eb = rest
            o = _relu(_dot(y, wet) + eb)
            for g in range(G):
                out_ref[g] = o[g * R:(g + 1) * R] if G > 1 else o
        else:
            f1t, f1b, f2t, f2b, f3t, f3b = rest
            h = _relu(_dot(y, f1t) + f1b)
            h = _relu(_dot(h, f2t) + f2b)
            lg = _dot(h, f3t) + f3b
            for g in range(G):
                lgg = lg[g * R:(g + 1) * R] if G > 1 else lg
                m = jnp.sum(lgg, axis=0, keepdims=True) * (1.0 / R)
                out_ref[g] = jnp.broadcast_to(m, (8, _NUM_CLASS))

    return body


def _layer_call(i, cfg, pts, ptsT, rep, fts1, params, sel=None, G=1):
    B = pts.shape[0]
    N = pts.shape[1]
    Nrep = rep.shape[1]
    fc = (i == len(_LAYER_CFG) - 1)
    l0 = (i == 0)
    emit_sel = (i == 0)
    R = 512 if Nrep == _N_PTS else Nrep
    T = Nrep // R
    Cf = fts1.shape[2]
    weights = _prep_weights(params, i, cfg, fc)
    if fc:
        out_shape = jax.ShapeDtypeStruct((B, 8, _NUM_CLASS), jnp.float32)
        out_spec = pl.BlockSpec((G, 8, _NUM_CLASS), lambda b, t: (b, 0, 0))
    else:
        C_next = _LAYER_CFG[i + 1][1] // 2
        out_shape = jax.ShapeDtypeStruct((B, Nrep, C_next), jnp.float32)
        out_spec = pl.BlockSpec((G, R, C_next), lambda b, t: (b, t, 0))
    if emit_sel:
        out_shape = (out_shape,
                     jax.ShapeDtypeStruct((B, N, _SEL_COLS), jnp.float32))
        out_spec = (out_spec,
                    pl.BlockSpec((1, R, _SEL_COLS), lambda b, t: (b, t, 0)))

    def _const(b, t):
        return (0, 0)

    if sel is not None:
        in_specs = [
            pl.BlockSpec((G, N, 3), lambda b, t: (b, 0, 0)),
            pl.BlockSpec((G, R, 3), lambda b, t: (b, t, 0)),
            pl.BlockSpec((G, N, Cf), lambda b, t: (b, 0, 0)),
            pl.BlockSpec((1, R, _SEL_COLS), lambda b, t: (b, t, 0)),
        ] + [pl.BlockSpec(w.shape, _const) for w in weights]
        args = (pts, rep, fts1, sel, *weights)
    else:
        in_specs = [
            pl.BlockSpec((G, 3, N), lambda b, t: (b, 0, 0)),
            pl.BlockSpec((G, N, 3), lambda b, t: (b, 0, 0)),
            pl.BlockSpec((G, R, 3), lambda b, t: (b, t, 0)),
            pl.BlockSpec((G, N, Cf), lambda b, t: (b, 0, 0)),
        ] + [pl.BlockSpec(w.shape, _const) for w in weights]
        args = (ptsT, pts, rep, fts1, *weights)

    body = _make_body(cfg, R, N, G, l0, fc, sel is not None, emit_sel)
    return pl.pallas_call(
        body,
        grid=(B // G, T),
        in_specs=in_specs,
        out_specs=out_spec,
        out_shape=out_shape,
    )(*args)


def kernel(pts, fts, params):
    pts = pts.astype(jnp.float32)
    ptsT = jnp.transpose(pts, (0, 2, 1))
    rep3 = pts[:, _SAMPLE_IDX, :]
    rep3T = jnp.transpose(rep3, (0, 2, 1))

    fts1, sel = _layer_call(0, _LAYER_CFG[0], pts, ptsT, pts,
                            fts.astype(jnp.float32), params)
    fts1 = _layer_call(1, _LAYER_CFG[1], pts, ptsT, pts, fts1, params, sel=sel)
    fts1 = _layer_call(2, _LAYER_CFG[2], pts, ptsT, pts, fts1, params, sel=sel)
    G = 8 if pts.shape[0] % 8 == 0 else 1
    fts1 = _layer_call(3, _LAYER_CFG[3], pts, ptsT, rep3, fts1, params, G=G)
    out = _layer_call(4, _LAYER_CFG[4], rep3, rep3T, rep3, fts1, params, G=G)
    return out[:, 0, :]


# speedup vs baseline: 14.2725x; 1.4042x over previous
import numpy as np
import jax
import jax.numpy as jnp
from jax.experimental import pallas as pl

_NUM_CLASS = 40
_N_PTS = 1024
_LAYER_CFG = [(3, 32, 8, 1, -1), (32, 64, 8, 2, -1), (64, 96, 8, 4, -1),
              (96, 128, 12, 4, 120), (128, 160, 12, 6, 120)]
_SAMPLE_IDX = np.random.RandomState(123).choice(_N_PTS, 120, replace=False)

_SEL_RANKS = 30
_SEL_COLS = 32


def _relu(x):
    return jnp.maximum(x, 0.0)


def _dot(a, b):
    return jax.lax.dot_general(a, b, (((1,), (0,)), ((), ())),
                               preferred_element_type=jnp.float32)


def _derived(cfg):
    C_in, C_out, K, D, _ = cfg
    Cmid, Cx = C_out // 4, C_out // 2
    Cm = Cmid + Cx
    dm = min(int(np.ceil(C_out / C_in)), 4)
    iters = (K - 1) * D + 2
    return Cmid, Cx, Cm, dm, iters


def _prep_weights(params, i, cfg, fc):
    p = params["pcnn%d" % i]
    C_in, C_out, K, D, _ = cfg
    Cmid, Cx, Cm, dm, _ = _derived(cfg)
    ws = [
        p["dense1"]["W"].T, p["dense1"]["b"][None],
        p["dense2"]["W"].T, p["dense2"]["b"][None],
        jnp.transpose(p["xconv_w"], (2, 1, 0)).reshape(3 * K, K * K),
        p["xconv_b"][None],
        p["xd1"]["W"].T, p["xd1"]["b"][None],
        p["xd2"]["W"].T, p["xd2"]["b"][None],
        jnp.einsum('km,jl->kjlm',
                   jnp.transpose(p["dw_w"], (2, 1, 0)).reshape(K, dm * Cm),
                   jnp.eye(K, dtype=jnp.float32)).reshape(K * K, K * dm * Cm),
        jnp.transpose(p["pw_w"].reshape(C_out, Cm, dm), (2, 1, 0)).reshape(dm * Cm, C_out),
        (p["dw_b"] @ p["pw_w"].T)[None],
        (p["bn_g"] / np.sqrt(1.0 + 1e-5))[None],
        p["bn_b"][None],
    ]
    if i == 0:
        ws += [p["dense"]["W"].T, p["dense"]["b"][None]]
    if not fc:
        nxt = params["pcnn%d" % (i + 1)]["dense"]
        ws += [nxt["W"].T, nxt["b"][None]]
    else:
        ws += [params["fc1"]["W"].T, params["fc1"]["b"][None],
               params["fc2"]["W"].T, params["fc2"]["b"][None],
               params["fc3"]["W"].T, params["fc3"]["b"][None]]
    return [w.astype(jnp.float32) for w in ws]


def _make_body(cfg, R, N, G, l0, fc, use_sel, emit_sel):
    C_in, C_out, K, D, _ = cfg
    Cmid, Cx, Cm, dm, iters = _derived(cfg)
    K2 = K * K
    RG = R * G
    n_ext = _SEL_RANKS if emit_sel else iters

    def body(*refs):
        nout = 2 if emit_sel else 1
        if use_sel:
            pts_r, rep_r, fts_r, idx_r = refs[:4]
        else:
            ptsT_r, pts_r, rep_r, fts_r = refs[:4]
        wv = [r[...] for r in refs[4:len(refs) - nout]]
        if emit_sel:
            out_ref, sel_ref = refs[-2], refs[-1]
        else:
            out_ref = refs[-1]
        (w1t, b1, w2t, b2, wxf, xb, xd1t, xb1, xd2t, xb2,
         bd, pwd, pwb, sv, bbv) = wv[:15]
        rest = wv[15:]
        if l0:
            w0t, b0 = rest[0], rest[1]
            rest = rest[2:]

        pf_g = []
        for g in range(G):
            if l0:
                pf_g.append(jnp.concatenate(
                    [_relu(_dot(fts_r[g], w0t) + b0), pts_r[g]], axis=1))
            else:
                pf_g.append(fts_r[g])

        iota = jax.lax.broadcasted_iota(jnp.int32, (RG, N), 1).astype(jnp.float32)
        rep = jnp.concatenate([rep_r[g] for g in range(G)], axis=0) if G > 1 \
            else rep_r[0]

        def gathers(oh):
            ys = []
            for g in range(G):
                ohg = oh[g * R:(g + 1) * R] if G > 1 else oh
                ys.append(_dot(ohg, pf_g[g]))
            y = jnp.concatenate(ys, 0) if G > 1 else ys[0]
            return y[:, Cx:Cx + 3], y[:, :Cx]

        p_list, f_list = [], []
        if use_sel:
            idxa = idx_r[0]
            for j in range(K):
                col = idxa[:, 1 + j * D:2 + j * D]
                oh = jnp.where(iota == col, 1.0, 0.0)
                pj, fj = gathers(oh)
                p_list.append(pj)
                f_list.append(fj)
        else:
            d2s = []
            for g in range(G):
                ptsT = ptsT_r[g]
                repg = rep_r[g]
                pts2 = jnp.sum(ptsT * ptsT, axis=0, keepdims=True)
                rep2 = jnp.sum(repg * repg, axis=1, keepdims=True)
                d2s.append(rep2 + pts2 - 2.0 * _dot(repg, ptsT))
            d2 = jnp.concatenate(d2s, axis=0) if G > 1 else d2s[0]
            big = jnp.float32(N)
            sel_cols = []
            for r in range(n_ext):
                vmin = jnp.min(d2, axis=1, keepdims=True)
                idxc = jnp.where(d2 == vmin, iota, big)
                fidx = jnp.min(idxc, axis=1, keepdims=True)
                onehot = idxc == fidx
                if emit_sel:
                    sel_cols.append(fidx)
                if 1 <= r <= 1 + (K - 1) * D and (r - 1) % D == 0:
                    oh = jnp.where(onehot, 1.0, 0.0)
                    pj, fj = gathers(oh)
                    p_list.append(pj)
                    f_list.append(fj)
                if r != n_ext - 1:
                    d2 = jnp.where(onehot, jnp.float32(np.inf), d2)
            if emit_sel:
                sel_cols.append(jnp.zeros((RG, _SEL_COLS - _SEL_RANKS), jnp.float32))
                sel_ref[0] = jnp.concatenate(sel_cols, axis=1)

        pl_j = [pj - rep for pj in p_list]
        PLs = jnp.concatenate(pl_j, axis=0)
        PLl = jnp.concatenate(pl_j, axis=1)

        fl = _relu(_dot(PLs, w1t) + b1)
        fl = _relu(_dot(fl, w2t) + b2)

        X = _relu(_dot(PLl, wxf) + xb)
        X = _relu(_dot(X, xd1t) + xb1)
        X = _dot(X, xd2t) + xb2

        cat = [jnp.concatenate([fl[j * RG:(j + 1) * RG], f_list[j]], axis=1)
               for j in range(K)]

        M = _dot(X, bd)
        acc = [jnp.zeros((RG, Cm), jnp.float32) for _ in range(dm)]
        for j in range(K):
            for dd in range(dm):
                o = (j * dm + dd) * Cm
                acc[dd] = acc[dd] + cat[j] * M[:, o:o + Cm]

        y = jnp.zeros((RG, C_out), jnp.float32) + pwb
        for dd in range(dm):
            y = y + _dot(acc[dd], pwd[dd * Cm:(dd + 1) * Cm, :])
        y = _relu(y) * sv + bbv

        if not fc:
            wet, eb = rest
            o = jnp.concatenate([_relu(_dot(y, wet) + eb), rep], axis=1)
            for g in range(G):
                out_ref[g] = o[g * R:(g + 1) * R] if G > 1 else o
        else:
            f1t, f1b, f2t, f2b, f3t, f3b = rest
            h = _relu(_dot(y, f1t) + f1b)
            h = _relu(_dot(h, f2t) + f2b)
            lg = _dot(h, f3t) + f3b
            for g in range(G):
                lgg = lg[g * R:(g + 1) * R] if G > 1 else lg
                m = jnp.sum(lgg, axis=0, keepdims=True) * (1.0 / R)
                out_ref[g] = jnp.broadcast_to(m, (8, _NUM_CLASS))

    return body


def _layer_call(i, cfg, pts, ptsT, rep, fts1, params, sel=None, G=1):
    B = pts.shape[0]
    N = pts.shape[1]
    Nrep = rep.shape[1]
    fc = (i == len(_LAYER_CFG) - 1)
    l0 = (i == 0)
    emit_sel = (i == 0)
    R = 512 if Nrep == _N_PTS else Nrep
    T = Nrep // R
    Cf = fts1.shape[2]
    weights = _prep_weights(params, i, cfg, fc)
    if fc:
        out_shape = jax.ShapeDtypeStruct((B, 8, _NUM_CLASS), jnp.float32)
        out_spec = pl.BlockSpec((G, 8, _NUM_CLASS), lambda b, t: (b, 0, 0))
    else:
        C_next = _LAYER_CFG[i + 1][1] // 2 + 3
        out_shape = jax.ShapeDtypeStruct((B, Nrep, C_next), jnp.float32)
        out_spec = pl.BlockSpec((G, R, C_next), lambda b, t: (b, t, 0))
    if emit_sel:
        out_shape = (out_shape,
                     jax.ShapeDtypeStruct((B, N, _SEL_COLS), jnp.float32))
        out_spec = (out_spec,
                    pl.BlockSpec((1, R, _SEL_COLS), lambda b, t: (b, t, 0)))

    def _const(b, t):
        return (0, 0)

    if sel is not None:
        in_specs = [
            pl.BlockSpec((G, N, 3), lambda b, t: (b, 0, 0)),
            pl.BlockSpec((G, R, 3), lambda b, t: (b, t, 0)),
            pl.BlockSpec((G, N, Cf), lambda b, t: (b, 0, 0)),
            pl.BlockSpec((1, R, _SEL_COLS), lambda b, t: (b, t, 0)),
        ] + [pl.BlockSpec(w.shape, _const) for w in weights]
        args = (pts, rep, fts1, sel, *weights)
    else:
        in_specs = [
            pl.BlockSpec((G, 3, N), lambda b, t: (b, 0, 0)),
            pl.BlockSpec((G, N, 3), lambda b, t: (b, 0, 0)),
            pl.BlockSpec((G, R, 3), lambda b, t: (b, t, 0)),
            pl.BlockSpec((G, N, Cf), lambda b, t: (b, 0, 0)),
        ] + [pl.BlockSpec(w.shape, _const) for w in weights]
        args = (ptsT, pts, rep, fts1, *weights)

    body = _make_body(cfg, R, N, G, l0, fc, sel is not None, emit_sel)
    return pl.pallas_call(
        body,
        grid=(B // G, T),
        in_specs=in_specs,
        out_specs=out_spec,
        out_shape=out_shape,
    )(*args)


def kernel(pts, fts, params):
    pts = pts.astype(jnp.float32)
    ptsT = jnp.transpose(pts, (0, 2, 1))
    rep3 = pts[:, _SAMPLE_IDX, :]
    rep3T = jnp.transpose(rep3, (0, 2, 1))

    fts1, sel = _layer_call(0, _LAYER_CFG[0], pts, ptsT, pts,
                            fts.astype(jnp.float32), params)
    fts1 = _layer_call(1, _LAYER_CFG[1], pts, ptsT, pts, fts1, params, sel=sel)
    fts1 = _layer_call(2, _LAYER_CFG[2], pts, ptsT, pts, fts1, params, sel=sel)
    G = 8 if pts.shape[0] % 8 == 0 else 1
    fts1 = _layer_call(3, _LAYER_CFG[3], pts, ptsT, rep3, fts1, params, G=G)
    out = _layer_call(4, _LAYER_CFG[4], rep3, rep3T, rep3, fts1, params, G=G)
    return out[:, 0, :]


# speedup vs baseline: 16.8539x; 1.1809x over previous
import numpy as np
import jax
import jax.numpy as jnp
from jax.experimental import pallas as pl

_NUM_CLASS = 40
_N_PTS = 1024
_LAYER_CFG = [(3, 32, 8, 1, -1), (32, 64, 8, 2, -1), (64, 96, 8, 4, -1),
              (96, 128, 12, 4, 120), (128, 160, 12, 6, 120)]
_SAMPLE_IDX = np.random.RandomState(123).choice(_N_PTS, 120, replace=False)

_SEL_RANKS = 30
_SEL_COLS = 32


def _relu(x):
    return jnp.maximum(x, 0.0)


def _dot(a, b):
    return jax.lax.dot_general(a, b, (((1,), (0,)), ((), ())),
                               preferred_element_type=jnp.float32)


def _derived(cfg):
    C_in, C_out, K, D, _ = cfg
    Cmid, Cx = C_out // 4, C_out // 2
    Cm = Cmid + Cx
    dm = min(int(np.ceil(C_out / C_in)), 4)
    iters = (K - 1) * D + 2
    return Cmid, Cx, Cm, dm, iters


def _prep_weights(params, i, cfg, fc):
    p = params["pcnn%d" % i]
    C_in, C_out, K, D, _ = cfg
    Cmid, Cx, Cm, dm, _ = _derived(cfg)
    ws = [
        p["dense1"]["W"].T, p["dense1"]["b"][None],
        p["dense2"]["W"].T, p["dense2"]["b"][None],
        jnp.transpose(p["xconv_w"], (2, 1, 0)).reshape(3 * K, K * K),
        p["xconv_b"][None],
        p["xd1"]["W"].T, p["xd1"]["b"][None],
        p["xd2"]["W"].T, p["xd2"]["b"][None],
        jnp.einsum('km,jl->kjlm',
                   jnp.transpose(p["dw_w"], (2, 1, 0)).reshape(K, dm * Cm),
                   jnp.eye(K, dtype=jnp.float32)).reshape(K * K, K * dm * Cm),
        jnp.transpose(p["pw_w"].reshape(C_out, Cm, dm), (2, 1, 0)).reshape(dm * Cm, C_out),
        (p["dw_b"] @ p["pw_w"].T)[None],
        (p["bn_g"] / np.sqrt(1.0 + 1e-5))[None],
        p["bn_b"][None],
    ]
    if i == 0:
        ws += [p["dense"]["W"].T, p["dense"]["b"][None]]
    if not fc:
        nxt = params["pcnn%d" % (i + 1)]["dense"]
        ws += [nxt["W"].T, nxt["b"][None]]
    else:
        ws += [params["fc1"]["W"].T, params["fc1"]["b"][None],
               params["fc2"]["W"].T, params["fc2"]["b"][None],
               params["fc3"]["W"].T, params["fc3"]["b"][None]]
    return [w.astype(jnp.float32) for w in ws]


def _make_body(cfg, R, N, G, l0, fc, use_sel, emit_sel):
    C_in, C_out, K, D, _ = cfg
    Cmid, Cx, Cm, dm, iters = _derived(cfg)
    K2 = K * K
    RG = R * G
    n_ext = _SEL_RANKS if emit_sel else iters

    def body(*refs):
        nout = 2 if emit_sel else 1
        if use_sel:
            pts_r, rep_r, fts_r, idx_r = refs[:4]
        else:
            ptsT_r, pts_r, rep_r, fts_r = refs[:4]
        wv = [r[...] for r in refs[4:len(refs) - nout]]
        if emit_sel:
            out_ref, sel_ref = refs[-2], refs[-1]
        else:
            out_ref = refs[-1]
        (w1t, b1, w2t, b2, wxf, xb, xd1t, xb1, xd2t, xb2,
         bd, pwd, pwb, sv, bbv) = wv[:15]
        rest = wv[15:]
        if l0:
            w0t, b0 = rest[0], rest[1]
            rest = rest[2:]

        pf_g = []
        for g in range(G):
            if l0:
                pf_g.append(jnp.concatenate(
                    [_relu(_dot(fts_r[g], w0t) + b0), pts_r[g]], axis=1))
            else:
                pf_g.append(fts_r[g])

        iota = jax.lax.broadcasted_iota(jnp.int32, (RG, N), 1)
        rep = jnp.concatenate([rep_r[g] for g in range(G)], axis=0) if G > 1 \
            else rep_r[0]

        def gathers(oh):
            ys = []
            for g in range(G):
                ohg = oh[g * R:(g + 1) * R] if G > 1 else oh
                ys.append(_dot(ohg, pf_g[g]))
            y = jnp.concatenate(ys, 0) if G > 1 else ys[0]
            return y[:, Cx:Cx + 3], y[:, :Cx]

        p_list, f_list = [], []
        if use_sel:
            idxa = idx_r[0]
            for j in range(K):
                col = idxa[:, 1 + j * D:2 + j * D]
                oh = jnp.where(iota == col, 1.0, 0.0)
                pj, fj = gathers(oh)
                p_list.append(pj)
                f_list.append(fj)
        else:
            d2s = []
            for g in range(G):
                ptsT = ptsT_r[g]
                repg = rep_r[g]
                pts2 = jnp.sum(ptsT * ptsT, axis=0, keepdims=True)
                rep2 = jnp.sum(repg * repg, axis=1, keepdims=True)
                d2s.append(rep2 + pts2 - 2.0 * _dot(repg, ptsT))
            d2 = jnp.concatenate(d2s, axis=0) if G > 1 else d2s[0]
            sel_cols = []
            for r in range(n_ext):
                fidx = jnp.argmin(d2, axis=1)[:, None]
                onehot = iota == fidx
                if emit_sel:
                    sel_cols.append(fidx)
                if 1 <= r <= 1 + (K - 1) * D and (r - 1) % D == 0:
                    oh = jnp.where(onehot, 1.0, 0.0)
                    pj, fj = gathers(oh)
                    p_list.append(pj)
                    f_list.append(fj)
                if r != n_ext - 1:
                    d2 = jnp.where(onehot, jnp.float32(np.inf), d2)
            if emit_sel:
                sel_cols.append(jnp.zeros((RG, _SEL_COLS - _SEL_RANKS), jnp.int32))
                sel_ref[0] = jnp.concatenate(sel_cols, axis=1)

        pl_j = [pj - rep for pj in p_list]
        PLs = jnp.concatenate(pl_j, axis=0)
        PLl = jnp.concatenate(pl_j, axis=1)

        fl = _relu(_dot(PLs, w1t) + b1)
        fl = _relu(_dot(fl, w2t) + b2)

        X = _relu(_dot(PLl, wxf) + xb)
        X = _relu(_dot(X, xd1t) + xb1)
        X = _dot(X, xd2t) + xb2

        cat = [jnp.concatenate([fl[j * RG:(j + 1) * RG], f_list[j]], axis=1)
               for j in range(K)]

        M = _dot(X, bd)
        acc = [jnp.zeros((RG, Cm), jnp.float32) for _ in range(dm)]
        for j in range(K):
            for dd in range(dm):
                o = (j * dm + dd) * Cm
                acc[dd] = acc[dd] + cat[j] * M[:, o:o + Cm]

        y = jnp.zeros((RG, C_out), jnp.float32) + pwb
        for dd in range(dm):
            y = y + _dot(acc[dd], pwd[dd * Cm:(dd + 1) * Cm, :])
        y = _relu(y) * sv + bbv

        if not fc:
            wet, eb = rest
            o = jnp.concatenate([_relu(_dot(y, wet) + eb), rep], axis=1)
            for g in range(G):
                out_ref[g] = o[g * R:(g + 1) * R] if G > 1 else o
        else:
            f1t, f1b, f2t, f2b, f3t, f3b = rest
            h = _relu(_dot(y, f1t) + f1b)
            h = _relu(_dot(h, f2t) + f2b)
            lg = _dot(h, f3t) + f3b
            for g in range(G):
                lgg = lg[g * R:(g + 1) * R] if G > 1 else lg
                m = jnp.sum(lgg, axis=0, keepdims=True) * (1.0 / R)
                out_ref[g] = jnp.broadcast_to(m, (8, _NUM_CLASS))

    return body


def _layer_call(i, cfg, pts, ptsT, rep, fts1, params, sel=None, G=1):
    B = pts.shape[0]
    N = pts.shape[1]
    Nrep = rep.shape[1]
    fc = (i == len(_LAYER_CFG) - 1)
    l0 = (i == 0)
    emit_sel = (i == 0)
    R = 512 if Nrep == _N_PTS else Nrep
    T = Nrep // R
    Cf = fts1.shape[2]
    weights = _prep_weights(params, i, cfg, fc)
    if fc:
        out_shape = jax.ShapeDtypeStruct((B, 8, _NUM_CLASS), jnp.float32)
        out_spec = pl.BlockSpec((G, 8, _NUM_CLASS), lambda b, t: (b, 0, 0))
    else:
        C_next = _LAYER_CFG[i + 1][1] // 2 + 3
        out_shape = jax.ShapeDtypeStruct((B, Nrep, C_next), jnp.float32)
        out_spec = pl.BlockSpec((G, R, C_next), lambda b, t: (b, t, 0))
    if emit_sel:
        out_shape = (out_shape,
                     jax.ShapeDtypeStruct((B, N, _SEL_COLS), jnp.int32))
        out_spec = (out_spec,
                    pl.BlockSpec((1, R, _SEL_COLS), lambda b, t: (b, t, 0)))

    def _const(b, t):
        return (0, 0)

    if sel is not None:
        in_specs = [
            pl.BlockSpec((G, N, 3), lambda b, t: (b, 0, 0)),
            pl.BlockSpec((G, R, 3), lambda b, t: (b, t, 0)),
            pl.BlockSpec((G, N, Cf), lambda b, t: (b, 0, 0)),
            pl.BlockSpec((1, R, _SEL_COLS), lambda b, t: (b, t, 0)),
        ] + [pl.BlockSpec(w.shape, _const) for w in weights]
        args = (pts, rep, fts1, sel, *weights)
    else:
        in_specs = [
            pl.BlockSpec((G, 3, N), lambda b, t: (b, 0, 0)),
            pl.BlockSpec((G, N, 3), lambda b, t: (b, 0, 0)),
            pl.BlockSpec((G, R, 3), lambda b, t: (b, t, 0)),
            pl.BlockSpec((G, N, Cf), lambda b, t: (b, 0, 0)),
        ] + [pl.BlockSpec(w.shape, _const) for w in weights]
        args = (ptsT, pts, rep, fts1, *weights)

    body = _make_body(cfg, R, N, G, l0, fc, sel is not None, emit_sel)
    return pl.pallas_call(
        body,
        grid=(B // G, T),
        in_specs=in_specs,
        out_specs=out_spec,
        out_shape=out_shape,
    )(*args)


def kernel(pts, fts, params):
    pts = pts.astype(jnp.float32)
    ptsT = jnp.transpose(pts, (0, 2, 1))
    rep3 = pts[:, _SAMPLE_IDX, :]
    rep3T = jnp.transpose(rep3, (0, 2, 1))

    fts1, sel = _layer_call(0, _LAYER_CFG[0], pts, ptsT, pts,
                            fts.astype(jnp.float32), params)
    fts1 = _layer_call(1, _LAYER_CFG[1], pts, ptsT, pts, fts1, params, sel=sel)
    fts1 = _layer_call(2, _LAYER_CFG[2], pts, ptsT, pts, fts1, params, sel=sel)
    G = 8 if pts.shape[0] % 8 == 0 else 1
    fts1 = _layer_call(3, _LAYER_CFG[3], pts, ptsT, rep3, fts1, params, G=G)
    out = _layer_call(4, _LAYER_CFG[4], rep3, rep3T, rep3, fts1, params, G=G)
    return out[:, 0, :]
